# baseline (speedup 1.0000x reference)
import functools

import jax
import jax.numpy as jnp
from jax import lax
from jax.experimental import pallas as pl
from jax.experimental.pallas import tpu as pltpu
from jax.experimental.pallas import tpu_sc as plsc

N = 10000
E = 160000
NUM_GRAPHS = 64
D_NODE = 128
G = 200
GP = 256
HW = 128
HV = HW // 16
NT = 10240
NU = 10240
EP = 163840
NW = 32
TE = EP // NW
EB = 128
NB = TE // EB
URW = 3072
RH = 2944
NRANGE = 4
RPS = URW // 16
CS = NT // 16

_MESH = plsc.VectorSubcoreMesh(core_axis_name="c", subcore_axis_name="s")
_SC_PARAMS = pltpu.CompilerParams(needs_layout_passes=False)
_NEG = -3.0e38
_f32 = jnp.float32


def _leaky(x):
    return jnp.where(x >= 0, x, 0.01 * x)


def _wid():
    return lax.axis_index("s") * 2 + lax.axis_index("c")


def _fill_vec(ref, n16, value):
    def body(i, _):
        ref[pl.ds(i * 16, 16)] = jnp.full((16,), value, jnp.float32)
        return 0
    lax.fori_loop(0, n16, body, 0)


def _fill2(ref, rows, cols, value):
    def body(i, _):
        for j in range(cols // 16):
            ref[i, pl.ds(16 * j, 16)] = jnp.full((16,), value, jnp.float32)
        return 0
    lax.fori_loop(0, rows, body, 0)


def _scatter_max(tbl, idx, val):
    old = plsc.load_gather(tbl, [idx])
    need0 = val > old

    def cond(need):
        return jnp.any(need)

    def body(need):
        plsc.store_scatter(tbl, [idx], val, mask=need)
        cur = plsc.load_gather(tbl, [idx])
        return jnp.logical_and(need, cur < val)

    lax.while_loop(cond, body, need0)


def _pass1_stage0_body(P2_hbm, Qp_hbm, q3_hbm, w23_hbm, srcg_hbm, dstl_hbm,
                       logit_out, mpart_out,
                       src_t, s2lo, s2hi, dst_t, q_t, m_t, w2_t, logit_t,
                       plo, phi, qlo, qhi, lred, sem):
    wid = _wid()
    lanes = lax.iota(jnp.int32, 16)
    bfly = [jnp.bitwise_xor(lanes, jnp.int32(sft)) for sft in (8, 4, 2, 1)]
    for p in range(3):
        pltpu.sync_copy(srcg_hbm.at[pl.ds((p * NW + wid) * TE, TE)], src_t)
        pltpu.sync_copy(dstl_hbm.at[p, wid], dst_t)
        pltpu.sync_copy(q3_hbm.at[pl.ds(p * NT, NT)], q_t)
        pltpu.sync_copy(w23_hbm.at[pl.ds(p * GP, GP)], w2_t)
        _fill_vec(m_t, NT // 16, _NEG)

        def pre(i, _):
            sv = src_t[pl.ds(i * 16, 16)]
            s2lo[pl.ds(i * 16, 16)] = sv * 2
            s2hi[pl.ds(i * 16, 16)] = sv * 2 + 1
            return 0
        lax.fori_loop(0, TE // 16, pre, 0)

        def blk(b, _):
            row0 = wid * TE + b * EB
            cp1 = pltpu.async_copy(P2_hbm.at[s2lo.at[pl.ds(b * EB, EB)]],
                                   plo, sem)
            cp2 = pltpu.async_copy(P2_hbm.at[s2hi.at[pl.ds(b * EB, EB)]],
                                   phi, sem)
            pltpu.sync_copy(Qp_hbm.at[p, 0, pl.ds(row0, EB)], qlo)
            pltpu.sync_copy(Qp_hbm.at[p, 1, pl.ds(row0, EB)], qhi)
            cp1.wait()
            cp2.wait()

            def grp(k, _):
                def edge(i, lv):
                    e = k * 16 + i
                    l_acc = jnp.zeros((16,), jnp.float32)
                    for j in range(HV):
                        h = _leaky(plo[e, pl.ds(16 * j, 16)]
                                   + qlo[e, pl.ds(16 * j, 16)])
                        l_acc = l_acc + h * w2_t[pl.ds(16 * j, 16)]
                    for j in range(HV):
                        h = _leaky(phi[e, pl.ds(16 * j, 16)]
                                   + qhi[e, pl.ds(16 * j, 16)])
                        l_acc = l_acc + h * w2_t[pl.ds(HW + 16 * j, 16)]
                    acc = l_acc
                    for idxv in bfly:
                        lred[pl.ds(0, 16)] = acc
                        acc = acc + plsc.load_gather(lred, [idxv])
                    return jnp.where(lanes == i, acc, lv)
                lv = lax.fori_loop(0, 16, edge, jnp.zeros((16,), jnp.float32))
                dv = dst_t[b, 0, pl.ds(16 * k, 16)]
                qd = plsc.load_gather(q_t, [dv])
                lg = _leaky(qd + lv)
                logit_t[pl.ds(b * EB + 16 * k, 16)] = lg
                _scatter_max(m_t, dv, lg)
                return 0
            lax.fori_loop(0, EB // 16, grp, 0)
            return 0
        lax.fori_loop(0, NB, blk, 0)

        pltpu.sync_copy(logit_t, logit_out.at[pl.ds(p * EP + wid * TE, TE)])
        pltpu.sync_copy(m_t, mpart_out.at[pl.ds((p * NW + wid) * NT, NT)])


def _pass1_layer_body(u3_hbm, v3_hbm, srcg_hbm, dstl_hbm,
                      logit_out, mpart_out,
                      src_t, dst_t, u_t, v_t, m_t, logit_t):
    wid = _wid()
    for p in range(3):
        pltpu.sync_copy(srcg_hbm.at[pl.ds((p * NW + wid) * TE, TE)], src_t)
        pltpu.sync_copy(dstl_hbm.at[p, wid], dst_t)
        pltpu.sync_copy(u3_hbm.at[pl.ds(p * NT, NT)], u_t)
        pltpu.sync_copy(v3_hbm.at[pl.ds(p * NT, NT)], v_t)
        _fill_vec(m_t, NT // 16, _NEG)

        def blk(b, _):
            for k in range(EB // 16):
                sv = src_t[pl.ds(b * EB + 16 * k, 16)] - p * N
                dv = dst_t[b, 0, pl.ds(16 * k, 16)]
                uu = plsc.load_gather(u_t, [dv])
                vv = plsc.load_gather(v_t, [sv])
                lg = _leaky(uu + vv)
                logit_t[pl.ds(b * EB + 16 * k, 16)] = lg
                _scatter_max(m_t, dv, lg)
            return 0
        lax.fori_loop(0, NB, blk, 0)

        pltpu.sync_copy(logit_t, logit_out.at[pl.ds(p * EP + wid * TE, TE)])
        pltpu.sync_copy(m_t, mpart_out.at[pl.ds((p * NW + wid) * NT, NT)])


def _pass2_common(p, is_stage0, rows_hbm, qp_hbm, logit_hbm, mpart_hbm,
                  dstl_hbm, u2_out,
                  src_t, s2_t, dst_t, dsth_t, logit_t, exbuf, m_t, mred,
                  pbuf, qbuf, zbuf, U_sh, m_sh, sem):
    wid = _wid()
    cid = lax.axis_index("c")
    sid = lax.axis_index("s")
    pltpu.sync_copy(dstl_hbm.at[p, wid], dst_t)
    pltpu.sync_copy(logit_hbm.at[pl.ds(p * EP + wid * TE, TE)], logit_t)

    for j in range(NW):
        pltpu.sync_copy(mpart_hbm.at[pl.ds((p * NW + j) * NT + sid * CS, CS)],
                        mred.at[j])

    def red(k, _):
        acc = mred[0, pl.ds(16 * k, 16)]
        for j in range(1, NW):
            acc = jnp.maximum(acc, mred[j, pl.ds(16 * k, 16)])
        m_t[pl.ds(16 * k, 16)] = acc
        return 0
    lax.fori_loop(0, CS // 16, red, 0)
    pltpu.sync_copy(m_t.at[pl.ds(0, CS)], m_sh.at[pl.ds(sid * CS, CS)])
    plsc.subcore_barrier()
    pltpu.sync_copy(m_sh, m_t)

    off = sid * RPS

    def h_loop(h, _):
        def dtr(b, _):
            for k in range(EB // 16):
                dv = dst_t[b, 0, pl.ds(16 * k, 16)]
                dvl = dv - h * RH
                ok = jnp.logical_and(dvl >= 0, dvl < RH)
                dsth_t[b, 0, pl.ds(16 * k, 16)] = jnp.where(ok, dvl, RH)
            return 0
        lax.fori_loop(0, NB, dtr, 0)

        def c_loop(c01, _):
            def pre(i, _):
                sv = src_t[pl.ds(i * 16, 16)]
                s2_t[pl.ds(i * 16, 16)] = sv * 2 + c01
                return 0
            lax.fori_loop(0, TE // 16, pre, 0)

            for r0 in range(0, RPS, EB):
                nr = min(EB, RPS - r0)
                pltpu.sync_copy(zbuf.at[pl.ds(0, nr)],
                                U_sh.at[pl.ds(off + r0, nr)])
            plsc.subcore_barrier()

            def blk(b, _):
                row0 = wid * TE + b * EB
                cp = pltpu.async_copy(rows_hbm.at[s2_t.at[pl.ds(b * EB, EB)]],
                                      pbuf, sem)

                @pl.when(is_stage0)
                def _():
                    pltpu.sync_copy(qp_hbm.at[p, c01, pl.ds(row0, EB)], qbuf)
                cp.wait()

                for k in range(EB // 16):
                    dv = dst_t[b, 0, pl.ds(16 * k, 16)]
                    md = plsc.load_gather(m_t, [dv])
                    lg = logit_t[pl.ds(b * EB + 16 * k, 16)]
                    exbuf[pl.ds(16 * k, 16)] = jnp.exp(lg - md)

                def edge(e, _):
                    sc = plsc.load_gather(exbuf,
                                          [jnp.full((16,), e, jnp.int32)])
                    for j in range(HV):
                        r = _leaky(pbuf[e, pl.ds(16 * j, 16)]
                                   + qbuf[e, pl.ds(16 * j, 16)])
                        pbuf[e, pl.ds(16 * j, 16)] = r * sc
                    return 0
                lax.fori_loop(0, EB, edge, 0)

                pltpu.sync_copy(pbuf, U_sh.at[dsth_t.at[b, 0]], add=True)
                return 0
            lax.fori_loop(0, NB, blk, 0)
            plsc.subcore_barrier()

            for r0 in range(0, RPS, EB):
                nr = min(EB, RPS - r0)
                pltpu.sync_copy(U_sh.at[pl.ds(off + r0, nr)],
                                u2_out.at[p, c01, h, cid,
                                          pl.ds(off + r0, nr)])
            plsc.subcore_barrier()
            return 0
        lax.fori_loop(0, 2, c_loop, 0)
        return 0
    lax.fori_loop(0, NRANGE, h_loop, 0)


def _pass2_body(rows_hbm, qp_hbm, flag_hbm, logit_hbm, mpart_hbm, srcg_hbm,
                dstl_hbm, u2_out,
                src_t, s2_t, dst_t, dsth_t, logit_t, exbuf, flag_t, m_t,
                mred, pbuf, qbuf, zbuf, U_sh, m_sh, sem):
    wid = _wid()
    _fill2(zbuf, EB, HW, 0.0)
    pltpu.sync_copy(flag_hbm, flag_t)
    fv = flag_t[pl.ds(0, 16)]
    is_stage0 = fv[0] == 1

    @pl.when(jnp.logical_not(is_stage0))
    def _():
        _fill2(qbuf, EB, HW, 0.0)

    for p in range(3):
        pltpu.sync_copy(srcg_hbm.at[pl.ds((p * NW + wid) * TE, TE)], src_t)
        _pass2_common(p, is_stage0, rows_hbm, qp_hbm, logit_hbm, mpart_hbm,
                      dstl_hbm, u2_out,
                      src_t, s2_t, dst_t, dsth_t, logit_t, exbuf, m_t, mred,
                      pbuf, qbuf, zbuf, U_sh, m_sh, sem)


@functools.partial(
    pl.kernel, mesh=_MESH, compiler_params=_SC_PARAMS,
    out_type=(jax.ShapeDtypeStruct((3 * EP,), _f32),
              jax.ShapeDtypeStruct((3 * NW * NT,), _f32)),
    scratch_types=[
        pltpu.VMEM((TE,), jnp.int32),
        pltpu.VMEM((TE,), jnp.int32),
        pltpu.VMEM((TE,), jnp.int32),
        pltpu.VMEM((NB, 1, EB), jnp.int32),
        pltpu.VMEM((NT,), _f32),
        pltpu.VMEM((NT,), _f32),
        pltpu.VMEM((GP,), _f32),
        pltpu.VMEM((TE,), _f32),
        pltpu.VMEM((EB, HW), _f32),
        pltpu.VMEM((EB, HW), _f32),
        pltpu.VMEM((EB, HW), _f32),
        pltpu.VMEM((EB, HW), _f32),
        pltpu.VMEM((128,), _f32),
        pltpu.SemaphoreType.DMA,
    ])
def _sc_pass1_stage0(*refs):
    _pass1_stage0_body(*refs)


@functools.partial(
    pl.kernel, mesh=_MESH, compiler_params=_SC_PARAMS,
    out_type=(jax.ShapeDtypeStruct((3 * EP,), _f32),
              jax.ShapeDtypeStruct((3 * NW * NT,), _f32)),
    scratch_types=[
        pltpu.VMEM((TE,), jnp.int32),
        pltpu.VMEM((NB, 1, EB), jnp.int32),
        pltpu.VMEM((NT,), _f32),
        pltpu.VMEM((NT,), _f32),
        pltpu.VMEM((NT,), _f32),
        pltpu.VMEM((TE,), _f32),
    ])
def _sc_pass1_layer_inner(*refs):
    _pass1_layer_body(*refs)


_sc_pass1_layer = jax.jit(_sc_pass1_layer_inner)


@functools.partial(
    pl.kernel, mesh=_MESH, compiler_params=_SC_PARAMS,
    out_type=jax.ShapeDtypeStruct((3, 2, NRANGE, 2, URW, HW), _f32),
    scratch_types=[
        pltpu.VMEM((TE,), jnp.int32),
        pltpu.VMEM((TE,), jnp.int32),
        pltpu.VMEM((NB, 1, EB), jnp.int32),
        pltpu.VMEM((NB, 1, EB), jnp.int32),
        pltpu.VMEM((TE,), _f32),
        pltpu.VMEM((EB,), _f32),
        pltpu.VMEM((16,), jnp.int32),
        pltpu.VMEM((NT,), _f32),
        pltpu.VMEM((NW, CS), _f32),
        pltpu.VMEM((EB, HW), _f32),
        pltpu.VMEM((EB, HW), _f32),
        pltpu.VMEM((EB, HW), _f32),
        pltpu.VMEM_SHARED((URW, HW), _f32),
        pltpu.VMEM_SHARED((NT,), _f32),
        pltpu.SemaphoreType.DMA,
    ])
def _sc_pass2_inner(*refs):
    _pass2_body(*refs)


_sc_pass2 = jax.jit(_sc_pass2_inner)


def _gru(x, h, Wih, Whh, bih, bhh):
    gi = x @ Wih.T + bih
    gh = h @ Whh.T + bhh
    ir, iz, inn = jnp.split(gi, 3, axis=-1)
    hr, hz, hn = jnp.split(gh, 3, axis=-1)
    r = jax.nn.sigmoid(ir + hr)
    z = jax.nn.sigmoid(iz + hz)
    n = jnp.tanh(inn + r * hn)
    return (1.0 - z) * n + z * h


def _seg_sum(x, seg, n):
    return jax.ops.segment_sum(x, seg, num_segments=n)


def _seg_max(x, seg, n):
    return jax.ops.segment_max(x, seg, num_segments=n)


def _agg_from_u2(U2p, W, b):
    Uc = U2p[:, :, 0] + U2p[:, :, 1]
    Un = jnp.concatenate([Uc[:, 0, :RH], Uc[:, 1, :RH], Uc[:, 2, :RH],
                          Uc[:, 3, :N - 3 * RH]], axis=1)
    s = Un[1, :, HW - 1]
    T = jnp.concatenate([Un[0], Un[1, :, :G - HW]], axis=1)
    has = s > 0.0
    sinv = jnp.where(has, 1.0 / jnp.where(has, s, 1.0), 0.0)
    C = T * sinv[:, None]
    return C @ W.T + has[:, None] * b, has


def _prep_edges(ei, p_idx):
    src = jnp.concatenate([ei[0], jnp.zeros((EP - E,), ei.dtype)])
    dst = jnp.concatenate([ei[1], jnp.full((EP - E,), N, ei.dtype)])
    srcg = (src + p_idx * N).astype(jnp.int32)
    dstl = dst.astype(jnp.int32).reshape(NW, NB, 1, EB)
    return srcg, dstl


def _pad_t(x):
    return jnp.concatenate([x, jnp.zeros((NT - N,), jnp.float32)])


def kernel(node_feats1, node_feats2, node_feats3, edge_feats1, edge_feats2,
           edge_feats3, edge_index1, edge_index2, edge_index3,
           node_graph_ids1, node_graph_ids2, node_graph_ids3,
           params1, params2, params3):
    xs = [node_feats1, node_feats2, node_feats3]
    efs = [edge_feats1, edge_feats2, edge_feats3]
    eis = [edge_index1, edge_index2, edge_index3]
    gids = [node_graph_ids1, node_graph_ids2, node_graph_ids3]
    ps = [params1, params2, params3]

    srcg = jnp.concatenate([_prep_edges(eis[p], p)[0] for p in range(3)])
    dstl = jnp.stack([_prep_edges(eis[p], p)[1] for p in range(3)])

    hvs, Ps, Qls, Qhs, qs, w2s = [], [], [], [], [], []
    for p in range(3):
        pr = ps[p]
        hv = _leaky(xs[p] @ pr['pn_W'].T + pr['pn_b'])
        P = xs[p] @ pr['pe1_W'][:, :D_NODE].T
        Pp = jnp.pad(P, ((0, 0), (0, GP - G)))
        ef = jnp.pad(efs[p], ((0, EP - E), (0, 0)))
        Q = ef @ pr['pe1_W'][:, D_NODE:].T + pr['pe1_b']
        Qp = jnp.pad(Q, ((0, 0), (0, GP - G)))
        Qp = Qp.at[:, GP - 1].set(1.0)
        q = hv @ pr['pe2_W'][0, :G] + pr['pe2_b'][0]
        w2 = jnp.pad(pr['pe2_W'][0, G:], (0, GP - G))
        hvs.append(hv); Ps.append(Pp)
        Qls.append(Qp[:, :HW]); Qhs.append(Qp[:, HW:])
        qs.append(_pad_t(q)); w2s.append(w2)

    P2 = jnp.concatenate(Ps).reshape(3 * N * 2, HW)
    Qp3 = jnp.stack([jnp.stack([Qls[p], Qhs[p]]) for p in range(3)])
    q3 = jnp.concatenate(qs)
    w23 = jnp.concatenate(w2s)

    logit, mpart = _sc_pass1_stage0(P2, Qp3, q3, w23, srcg, dstl)
    flag1 = jnp.ones((16,), jnp.int32)
    flag0 = jnp.zeros((16,), jnp.int32)
    U2 = _sc_pass2(P2, Qp3, flag1, logit, mpart, srcg, dstl)

    nodes = []
    for p in range(3):
        pr = ps[p]
        ctx_lin, _ = _agg_from_u2(U2[p], pr['ag1_et_W'], pr['ag1_et_b'])
        ctx = jax.nn.elu(ctx_lin)
        node = jax.nn.relu(_gru(ctx, hvs[p], pr['ag1_Wih'], pr['ag1_Whh'],
                                pr['ag1_bih'], pr['ag1_bhh']))
        nodes.append(node)

    for li in range(2):
        us, vs, ngs = [], [], []
        for p in range(3):
            lp = ps[p]['layers'][li]
            u = nodes[p] @ lp['pe_W'][0, :G] + lp['pe_b'][0]
            v = nodes[p] @ lp['pe_W'][0, G:]
            ng = jnp.pad(nodes[p], ((0, 0), (0, GP - G)))
            ng = ng.at[:, GP - 1].set(1.0)
            us.append(_pad_t(u)); vs.append(_pad_t(v)); ngs.append(ng)
        u3 = jnp.concatenate(us)
        v3 = jnp.concatenate(vs)
        ng2 = jnp.concatenate(ngs).reshape(3 * N * 2, HW)

        logit, mpart = _sc_pass1_layer(u3, v3, srcg, dstl)
        U2 = _sc_pass2(ng2, Qp3, flag0, logit, mpart, srcg, dstl)

        new_nodes = []
        for p in range(3):
            lp = ps[p]['layers'][li]
            c_lin, _ = _agg_from_u2(U2[p], lp['pn_W'], lp['pn_b'])
            node = jax.nn.relu(_gru(jax.nn.elu(c_lin), nodes[p],
                                    lp['Wih'], lp['Whh'], lp['bih'], lp['bhh']))
            new_nodes.append(node)
        nodes = new_nodes

    gs = []
    for p in range(3):
        pr = ps[p]
        node = nodes[p]
        gid = gids[p]
        oh = (gid[:, None] == jnp.arange(NUM_GRAPHS)[None, :]).astype(jnp.float32)
        g_feats = oh.T @ node
        for rp in pr['readout']:
            rw1 = rp['cl_W'][0, :G]
            rw2 = rp['cl_W'][0, G:]
            zg = jax.nn.relu(g_feats) @ rw1
            z = _leaky(oh @ zg + node @ rw2 + rp['cl_b'][0])
            mg = jnp.max(jnp.where(oh > 0, z[:, None], -jnp.inf), axis=0)
            mg = jnp.where(jnp.isfinite(mg), mg, 0.0)
            exn = jnp.exp(z - oh @ mg)
            sg = oh.T @ exn
            Tg = oh.T @ (exn[:, None] * node)
            hasg = sg > 0.0
            sginv = jnp.where(hasg, 1.0 / jnp.where(hasg, sg, 1.0), 0.0)
            g_repr = (Tg * sginv[:, None]) @ rp['prn_W'].T + hasg[:, None] * rp['prn_b']
            g_feats = jax.nn.relu(_gru(jax.nn.elu(g_repr), g_feats,
                                       rp['Wih'], rp['Whh'], rp['bih'], rp['bhh']))
        gs.append(g_feats)

    return _predict_head(gs[0], gs[1], gs[2], params1, params2, params3)


def _head_body(g1_ref, g2_ref, g3_ref, w_ref, b_ref, o_ref):
    gcat = jnp.concatenate([g1_ref[...], g2_ref[...], g3_ref[...]], axis=1)
    o_ref[...] = gcat @ w_ref[...] + b_ref[...]


def _predict_head(g1, g2, g3, p1, p2, p3):
    w = jnp.zeros((3 * G, 128), jnp.float32)
    w = w.at[:G, 0].set(p1['pred_W'][0])
    w = w.at[G:2 * G, 1].set(p2['pred_W'][0])
    w = w.at[2 * G:, 2].set(p3['pred_W'][0])
    b = jnp.zeros((1, 128), jnp.float32)
    b = b.at[0, 0].set(p1['pred_b'][0]).at[0, 1].set(p2['pred_b'][0]).at[0, 2].set(p3['pred_b'][0])
    o = pl.pallas_call(
        _head_body,
        out_shape=jax.ShapeDtypeStruct((NUM_GRAPHS, 128), jnp.float32),
    )(g1, g2, g3, w, b)
    return o[:, :3]


# speedup vs baseline: 1.0160x; 1.0160x over previous
import functools

import jax
import jax.numpy as jnp
from jax import lax
from jax.experimental import pallas as pl
from jax.experimental.pallas import tpu as pltpu
from jax.experimental.pallas import tpu_sc as plsc

N = 10000
E = 160000
NUM_GRAPHS = 64
D_NODE = 128
G = 200
GP = 256
HW = 128
HV = HW // 16
NT = 10240
NU = 10240
EP = 163840
NW = 32
TE = EP // NW
EB = 128
NB = TE // EB
URW = 3072
RH = 2944
NRANGE = 4
RPS = URW // 16
CS = NT // 16

_MESH = plsc.VectorSubcoreMesh(core_axis_name="c", subcore_axis_name="s")
_SC_PARAMS = pltpu.CompilerParams(needs_layout_passes=False)
_NEG = -3.0e38
_f32 = jnp.float32


def _leaky(x):
    return jnp.where(x >= 0, x, 0.01 * x)


def _wid():
    return lax.axis_index("s") * 2 + lax.axis_index("c")


def _fill_vec(ref, n16, value):
    def body(i, _):
        ref[pl.ds(i * 16, 16)] = jnp.full((16,), value, jnp.float32)
        return 0
    lax.fori_loop(0, n16, body, 0)


def _fill2(ref, rows, cols, value):
    def body(i, _):
        for j in range(cols // 16):
            ref[i, pl.ds(16 * j, 16)] = jnp.full((16,), value, jnp.float32)
        return 0
    lax.fori_loop(0, rows, body, 0)


def _scatter_max(tbl, idx, val):
    old = plsc.load_gather(tbl, [idx])
    need0 = val > old

    def cond(need):
        return jnp.any(need)

    def body(need):
        plsc.store_scatter(tbl, [idx], val, mask=need)
        cur = plsc.load_gather(tbl, [idx])
        return jnp.logical_and(need, cur < val)

    lax.while_loop(cond, body, need0)


def _pass1_stage0_body(P2_hbm, Qp_hbm, q3_hbm, w23_hbm, srcg_hbm, dstl_hbm,
                       logit_out, mpart_out,
                       src_t, s2lo, s2hi, dst_t, q_t, m_t, w2_t, logit_t,
                       plo, phi, qlo, qhi, lred, sem):
    wid = _wid()
    lanes = lax.iota(jnp.int32, 16)
    bfly = [jnp.bitwise_xor(lanes, jnp.int32(sft)) for sft in (8, 4, 2, 1)]
    for p in range(3):
        pltpu.sync_copy(srcg_hbm.at[pl.ds((p * NW + wid) * TE, TE)], src_t)
        pltpu.sync_copy(dstl_hbm.at[p, wid], dst_t)
        pltpu.sync_copy(q3_hbm.at[pl.ds(p * NT, NT)], q_t)
        pltpu.sync_copy(w23_hbm.at[pl.ds(p * GP, GP)], w2_t)
        _fill_vec(m_t, NT // 16, _NEG)

        def pre(i, _):
            sv = src_t[pl.ds(i * 16, 16)]
            s2lo[pl.ds(i * 16, 16)] = sv * 2
            s2hi[pl.ds(i * 16, 16)] = sv * 2 + 1
            return 0
        lax.fori_loop(0, TE // 16, pre, 0)

        def blk(b, _):
            row0 = wid * TE + b * EB
            cp1 = pltpu.async_copy(P2_hbm.at[s2lo.at[pl.ds(b * EB, EB)]],
                                   plo, sem)
            cp2 = pltpu.async_copy(P2_hbm.at[s2hi.at[pl.ds(b * EB, EB)]],
                                   phi, sem)
            pltpu.sync_copy(Qp_hbm.at[p, 0, pl.ds(row0, EB)], qlo)
            pltpu.sync_copy(Qp_hbm.at[p, 1, pl.ds(row0, EB)], qhi)
            cp1.wait()
            cp2.wait()

            def grp(k, _):
                def edge(i, lv):
                    e = k * 16 + i
                    l_acc = jnp.zeros((16,), jnp.float32)
                    for j in range(HV):
                        h = _leaky(plo[e, pl.ds(16 * j, 16)]
                                   + qlo[e, pl.ds(16 * j, 16)])
                        l_acc = l_acc + h * w2_t[pl.ds(16 * j, 16)]
                    for j in range(HV):
                        h = _leaky(phi[e, pl.ds(16 * j, 16)]
                                   + qhi[e, pl.ds(16 * j, 16)])
                        l_acc = l_acc + h * w2_t[pl.ds(HW + 16 * j, 16)]
                    acc = l_acc
                    for idxv in bfly:
                        lred[pl.ds(0, 16)] = acc
                        acc = acc + plsc.load_gather(lred, [idxv])
                    return jnp.where(lanes == i, acc, lv)
                lv = lax.fori_loop(0, 16, edge, jnp.zeros((16,), jnp.float32))
                dv = dst_t[b, 0, pl.ds(16 * k, 16)]
                qd = plsc.load_gather(q_t, [dv])
                lg = _leaky(qd + lv)
                logit_t[pl.ds(b * EB + 16 * k, 16)] = lg
                _scatter_max(m_t, dv, lg)
                return 0
            lax.fori_loop(0, EB // 16, grp, 0)
            return 0
        lax.fori_loop(0, NB, blk, 0)

        pltpu.sync_copy(logit_t, logit_out.at[pl.ds(p * EP + wid * TE, TE)])
        pltpu.sync_copy(m_t, mpart_out.at[pl.ds((p * NW + wid) * NT, NT)])


def _pass1_layer_body(u3_hbm, v3_hbm, srcg_hbm, dstl_hbm,
                      logit_out, mpart_out,
                      src_t, dst_t, u_t, v_t, m_t, logit_t):
    wid = _wid()
    for p in range(3):
        pltpu.sync_copy(srcg_hbm.at[pl.ds((p * NW + wid) * TE, TE)], src_t)
        pltpu.sync_copy(dstl_hbm.at[p, wid], dst_t)
        pltpu.sync_copy(u3_hbm.at[pl.ds(p * NT, NT)], u_t)
        pltpu.sync_copy(v3_hbm.at[pl.ds(p * NT, NT)], v_t)
        _fill_vec(m_t, NT // 16, _NEG)

        def blk(b, _):
            for k in range(EB // 16):
                sv = src_t[pl.ds(b * EB + 16 * k, 16)] - p * N
                dv = dst_t[b, 0, pl.ds(16 * k, 16)]
                uu = plsc.load_gather(u_t, [dv])
                vv = plsc.load_gather(v_t, [sv])
                lg = _leaky(uu + vv)
                logit_t[pl.ds(b * EB + 16 * k, 16)] = lg
                _scatter_max(m_t, dv, lg)
            return 0
        lax.fori_loop(0, NB, blk, 0)

        pltpu.sync_copy(logit_t, logit_out.at[pl.ds(p * EP + wid * TE, TE)])
        pltpu.sync_copy(m_t, mpart_out.at[pl.ds((p * NW + wid) * NT, NT)])


def _pass2_common(p, is_stage0, rows_hbm, qp_hbm, logit_hbm, mpart_hbm,
                  dstl_hbm, u2_out,
                  src_t, s2_t, dst_t, dsth_t, logit_t, exbuf, m_t, mred,
                  pbuf, qbuf, zbuf, U_sh, m_sh, sem):
    wid = _wid()
    cid = lax.axis_index("c")
    sid = lax.axis_index("s")
    pltpu.sync_copy(dstl_hbm.at[p, wid], dst_t)
    pltpu.sync_copy(logit_hbm.at[pl.ds(p * EP + wid * TE, TE)], logit_t)

    for j in range(NW):
        pltpu.sync_copy(mpart_hbm.at[pl.ds((p * NW + j) * NT + sid * CS, CS)],
                        mred.at[j])

    def red(k, _):
        acc = mred[0, pl.ds(16 * k, 16)]
        for j in range(1, NW):
            acc = jnp.maximum(acc, mred[j, pl.ds(16 * k, 16)])
        m_t[pl.ds(16 * k, 16)] = acc
        return 0
    lax.fori_loop(0, CS // 16, red, 0)
    pltpu.sync_copy(m_t.at[pl.ds(0, CS)], m_sh.at[pl.ds(sid * CS, CS)])
    plsc.subcore_barrier()
    pltpu.sync_copy(m_sh, m_t)

    off = sid * RPS

    def h_loop(h, _):
        def dtr(b, _):
            for k in range(EB // 16):
                dv = dst_t[b, 0, pl.ds(16 * k, 16)]
                dvl = dv - h * RH
                ok = jnp.logical_and(dvl >= 0, dvl < RH)
                dsth_t[b, 0, pl.ds(16 * k, 16)] = jnp.where(ok, dvl, RH)
            return 0
        lax.fori_loop(0, NB, dtr, 0)

        def c_loop(c01, _):
            def pre(i, _):
                sv = src_t[pl.ds(i * 16, 16)]
                s2_t[pl.ds(i * 16, 16)] = sv * 2 + c01
                return 0
            lax.fori_loop(0, TE // 16, pre, 0)

            for r0 in range(0, RPS, EB):
                nr = min(EB, RPS - r0)
                pltpu.sync_copy(zbuf.at[pl.ds(0, nr)],
                                U_sh.at[pl.ds(off + r0, nr)])
            plsc.subcore_barrier()

            def blk(b, _):
                fvec = dst_t[b, 0, pl.ds(0, 16)]
                lvec = dst_t[b, 0, pl.ds(EB - 16, 16)]
                hit = jnp.logical_and(fvec[0] // RH <= h,
                                      lvec[15] // RH >= h)

                @pl.when(hit)
                def _():
                    row0 = wid * TE + b * EB
                    cp = pltpu.async_copy(
                        rows_hbm.at[s2_t.at[pl.ds(b * EB, EB)]], pbuf, sem)

                    @pl.when(is_stage0)
                    def _():
                        pltpu.sync_copy(qp_hbm.at[p, c01, pl.ds(row0, EB)],
                                        qbuf)
                    cp.wait()

                    for k in range(EB // 16):
                        dv = dst_t[b, 0, pl.ds(16 * k, 16)]
                        md = plsc.load_gather(m_t, [dv])
                        lg = logit_t[pl.ds(b * EB + 16 * k, 16)]
                        exbuf[pl.ds(16 * k, 16)] = jnp.exp(lg - md)

                    def edge(e, _):
                        sc = plsc.load_gather(exbuf,
                                              [jnp.full((16,), e, jnp.int32)])
                        for j in range(HV):
                            r = _leaky(pbuf[e, pl.ds(16 * j, 16)]
                                       + qbuf[e, pl.ds(16 * j, 16)])
                            pbuf[e, pl.ds(16 * j, 16)] = r * sc
                        return 0
                    lax.fori_loop(0, EB, edge, 0)

                    pltpu.sync_copy(pbuf, U_sh.at[dsth_t.at[b, 0]], add=True)
                return 0
            lax.fori_loop(0, NB, blk, 0)
            plsc.subcore_barrier()

            for r0 in range(0, RPS, EB):
                nr = min(EB, RPS - r0)
                pltpu.sync_copy(U_sh.at[pl.ds(off + r0, nr)],
                                u2_out.at[p, c01, h, cid,
                                          pl.ds(off + r0, nr)])
            plsc.subcore_barrier()
            return 0
        lax.fori_loop(0, 2, c_loop, 0)
        return 0
    lax.fori_loop(0, NRANGE, h_loop, 0)


def _pass2_body(rows_hbm, qp_hbm, flag_hbm, logit_hbm, mpart_hbm, srcg_hbm,
                dstl_hbm, u2_out,
                src_t, s2_t, dst_t, dsth_t, logit_t, exbuf, flag_t, m_t,
                mred, pbuf, qbuf, zbuf, U_sh, m_sh, sem):
    wid = _wid()
    _fill2(zbuf, EB, HW, 0.0)
    pltpu.sync_copy(flag_hbm, flag_t)
    fv = flag_t[pl.ds(0, 16)]
    is_stage0 = fv[0] == 1

    @pl.when(jnp.logical_not(is_stage0))
    def _():
        _fill2(qbuf, EB, HW, 0.0)

    for p in range(3):
        pltpu.sync_copy(srcg_hbm.at[pl.ds((p * NW + wid) * TE, TE)], src_t)
        _pass2_common(p, is_stage0, rows_hbm, qp_hbm, logit_hbm, mpart_hbm,
                      dstl_hbm, u2_out,
                      src_t, s2_t, dst_t, dsth_t, logit_t, exbuf, m_t, mred,
                      pbuf, qbuf, zbuf, U_sh, m_sh, sem)


@functools.partial(
    pl.kernel, mesh=_MESH, compiler_params=_SC_PARAMS,
    out_type=(jax.ShapeDtypeStruct((3 * EP,), _f32),
              jax.ShapeDtypeStruct((3 * NW * NT,), _f32)),
    scratch_types=[
        pltpu.VMEM((TE,), jnp.int32),
        pltpu.VMEM((TE,), jnp.int32),
        pltpu.VMEM((TE,), jnp.int32),
        pltpu.VMEM((NB, 1, EB), jnp.int32),
        pltpu.VMEM((NT,), _f32),
        pltpu.VMEM((NT,), _f32),
        pltpu.VMEM((GP,), _f32),
        pltpu.VMEM((TE,), _f32),
        pltpu.VMEM((EB, HW), _f32),
        pltpu.VMEM((EB, HW), _f32),
        pltpu.VMEM((EB, HW), _f32),
        pltpu.VMEM((EB, HW), _f32),
        pltpu.VMEM((128,), _f32),
        pltpu.SemaphoreType.DMA,
    ])
def _sc_pass1_stage0(*refs):
    _pass1_stage0_body(*refs)


@functools.partial(
    pl.kernel, mesh=_MESH, compiler_params=_SC_PARAMS,
    out_type=(jax.ShapeDtypeStruct((3 * EP,), _f32),
              jax.ShapeDtypeStruct((3 * NW * NT,), _f32)),
    scratch_types=[
        pltpu.VMEM((TE,), jnp.int32),
        pltpu.VMEM((NB, 1, EB), jnp.int32),
        pltpu.VMEM((NT,), _f32),
        pltpu.VMEM((NT,), _f32),
        pltpu.VMEM((NT,), _f32),
        pltpu.VMEM((TE,), _f32),
    ])
def _sc_pass1_layer_inner(*refs):
    _pass1_layer_body(*refs)


_sc_pass1_layer = jax.jit(_sc_pass1_layer_inner)


@functools.partial(
    pl.kernel, mesh=_MESH, compiler_params=_SC_PARAMS,
    out_type=jax.ShapeDtypeStruct((3, 2, NRANGE, 2, URW, HW), _f32),
    scratch_types=[
        pltpu.VMEM((TE,), jnp.int32),
        pltpu.VMEM((TE,), jnp.int32),
        pltpu.VMEM((NB, 1, EB), jnp.int32),
        pltpu.VMEM((NB, 1, EB), jnp.int32),
        pltpu.VMEM((TE,), _f32),
        pltpu.VMEM((EB,), _f32),
        pltpu.VMEM((16,), jnp.int32),
        pltpu.VMEM((NT,), _f32),
        pltpu.VMEM((NW, CS), _f32),
        pltpu.VMEM((EB, HW), _f32),
        pltpu.VMEM((EB, HW), _f32),
        pltpu.VMEM((EB, HW), _f32),
        pltpu.VMEM_SHARED((URW, HW), _f32),
        pltpu.VMEM_SHARED((NT,), _f32),
        pltpu.SemaphoreType.DMA,
    ])
def _sc_pass2_inner(*refs):
    _pass2_body(*refs)


_sc_pass2 = jax.jit(_sc_pass2_inner)


def _gru(x, h, Wih, Whh, bih, bhh):
    gi = x @ Wih.T + bih
    gh = h @ Whh.T + bhh
    ir, iz, inn = jnp.split(gi, 3, axis=-1)
    hr, hz, hn = jnp.split(gh, 3, axis=-1)
    r = jax.nn.sigmoid(ir + hr)
    z = jax.nn.sigmoid(iz + hz)
    n = jnp.tanh(inn + r * hn)
    return (1.0 - z) * n + z * h


def _seg_sum(x, seg, n):
    return jax.ops.segment_sum(x, seg, num_segments=n)


def _seg_max(x, seg, n):
    return jax.ops.segment_max(x, seg, num_segments=n)


def _agg_from_u2(U2p, W, b):
    Uc = U2p[:, :, 0] + U2p[:, :, 1]
    Un = jnp.concatenate([Uc[:, 0, :RH], Uc[:, 1, :RH], Uc[:, 2, :RH],
                          Uc[:, 3, :N - 3 * RH]], axis=1)
    s = Un[1, :, HW - 1]
    T = jnp.concatenate([Un[0], Un[1, :, :G - HW]], axis=1)
    has = s > 0.0
    sinv = jnp.where(has, 1.0 / jnp.where(has, s, 1.0), 0.0)
    C = T * sinv[:, None]
    return C @ W.T + has[:, None] * b, has


def _prep_edges(ei, p_idx):
    src = jnp.concatenate([ei[0], jnp.zeros((EP - E,), ei.dtype)])
    dst = jnp.concatenate([ei[1], jnp.full((EP - E,), N, ei.dtype)])
    perm = jnp.argsort((dst // RH).astype(jnp.int32), stable=True)
    src = src[perm]
    dst = dst[perm]
    srcg = (src + p_idx * N).astype(jnp.int32)
    dstl = dst.astype(jnp.int32).reshape(NW, NB, 1, EB)
    return srcg, dstl, perm


def _pad_t(x):
    return jnp.concatenate([x, jnp.zeros((NT - N,), jnp.float32)])


def kernel(node_feats1, node_feats2, node_feats3, edge_feats1, edge_feats2,
           edge_feats3, edge_index1, edge_index2, edge_index3,
           node_graph_ids1, node_graph_ids2, node_graph_ids3,
           params1, params2, params3):
    xs = [node_feats1, node_feats2, node_feats3]
    efs = [edge_feats1, edge_feats2, edge_feats3]
    eis = [edge_index1, edge_index2, edge_index3]
    gids = [node_graph_ids1, node_graph_ids2, node_graph_ids3]
    ps = [params1, params2, params3]

    preps = [_prep_edges(eis[p], p) for p in range(3)]
    srcg = jnp.concatenate([pr[0] for pr in preps])
    dstl = jnp.stack([pr[1] for pr in preps])
    perms = [pr[2] for pr in preps]

    hvs, Ps, Qls, Qhs, qs, w2s = [], [], [], [], [], []
    for p in range(3):
        pr = ps[p]
        hv = _leaky(xs[p] @ pr['pn_W'].T + pr['pn_b'])
        P = xs[p] @ pr['pe1_W'][:, :D_NODE].T
        Pp = jnp.pad(P, ((0, 0), (0, GP - G)))
        ef = jnp.pad(efs[p], ((0, EP - E), (0, 0)))[perms[p]]
        Q = ef @ pr['pe1_W'][:, D_NODE:].T + pr['pe1_b']
        Qp = jnp.pad(Q, ((0, 0), (0, GP - G)))
        Qp = Qp.at[:, GP - 1].set(1.0)
        q = hv @ pr['pe2_W'][0, :G] + pr['pe2_b'][0]
        w2 = jnp.pad(pr['pe2_W'][0, G:], (0, GP - G))
        hvs.append(hv); Ps.append(Pp)
        Qls.append(Qp[:, :HW]); Qhs.append(Qp[:, HW:])
        qs.append(_pad_t(q)); w2s.append(w2)

    P2 = jnp.concatenate(Ps).reshape(3 * N * 2, HW)
    Qp3 = jnp.stack([jnp.stack([Qls[p], Qhs[p]]) for p in range(3)])
    q3 = jnp.concatenate(qs)
    w23 = jnp.concatenate(w2s)

    logit, mpart = _sc_pass1_stage0(P2, Qp3, q3, w23, srcg, dstl)
    flag1 = jnp.ones((16,), jnp.int32)
    flag0 = jnp.zeros((16,), jnp.int32)
    U2 = _sc_pass2(P2, Qp3, flag1, logit, mpart, srcg, dstl)

    nodes = []
    for p in range(3):
        pr = ps[p]
        ctx_lin, _ = _agg_from_u2(U2[p], pr['ag1_et_W'], pr['ag1_et_b'])
        ctx = jax.nn.elu(ctx_lin)
        node = jax.nn.relu(_gru(ctx, hvs[p], pr['ag1_Wih'], pr['ag1_Whh'],
                                pr['ag1_bih'], pr['ag1_bhh']))
        nodes.append(node)

    for li in range(2):
        us, vs, ngs = [], [], []
        for p in range(3):
            lp = ps[p]['layers'][li]
            u = nodes[p] @ lp['pe_W'][0, :G] + lp['pe_b'][0]
            v = nodes[p] @ lp['pe_W'][0, G:]
            ng = jnp.pad(nodes[p], ((0, 0), (0, GP - G)))
            ng = ng.at[:, GP - 1].set(1.0)
            us.append(_pad_t(u)); vs.append(_pad_t(v)); ngs.append(ng)
        u3 = jnp.concatenate(us)
        v3 = jnp.concatenate(vs)
        ng2 = jnp.concatenate(ngs).reshape(3 * N * 2, HW)

        logit, mpart = _sc_pass1_layer(u3, v3, srcg, dstl)
        U2 = _sc_pass2(ng2, Qp3, flag0, logit, mpart, srcg, dstl)

        new_nodes = []
        for p in range(3):
            lp = ps[p]['layers'][li]
            c_lin, _ = _agg_from_u2(U2[p], lp['pn_W'], lp['pn_b'])
            node = jax.nn.relu(_gru(jax.nn.elu(c_lin), nodes[p],
                                    lp['Wih'], lp['Whh'], lp['bih'], lp['bhh']))
            new_nodes.append(node)
        nodes = new_nodes

    gs = []
    for p in range(3):
        pr = ps[p]
        node = nodes[p]
        gid = gids[p]
        oh = (gid[:, None] == jnp.arange(NUM_GRAPHS)[None, :]).astype(jnp.float32)
        g_feats = oh.T @ node
        for rp in pr['readout']:
            rw1 = rp['cl_W'][0, :G]
            rw2 = rp['cl_W'][0, G:]
            zg = jax.nn.relu(g_feats) @ rw1
            z = _leaky(oh @ zg + node @ rw2 + rp['cl_b'][0])
            mg = jnp.max(jnp.where(oh > 0, z[:, None], -jnp.inf), axis=0)
            mg = jnp.where(jnp.isfinite(mg), mg, 0.0)
            exn = jnp.exp(z - oh @ mg)
            sg = oh.T @ exn
            Tg = oh.T @ (exn[:, None] * node)
            hasg = sg > 0.0
            sginv = jnp.where(hasg, 1.0 / jnp.where(hasg, sg, 1.0), 0.0)
            g_repr = (Tg * sginv[:, None]) @ rp['prn_W'].T + hasg[:, None] * rp['prn_b']
            g_feats = jax.nn.relu(_gru(jax.nn.elu(g_repr), g_feats,
                                       rp['Wih'], rp['Whh'], rp['bih'], rp['bhh']))
        gs.append(g_feats)

    return _predict_head(gs[0], gs[1], gs[2], params1, params2, params3)


def _head_body(g1_ref, g2_ref, g3_ref, w_ref, b_ref, o_ref):
    gcat = jnp.concatenate([g1_ref[...], g2_ref[...], g3_ref[...]], axis=1)
    o_ref[...] = gcat @ w_ref[...] + b_ref[...]


def _predict_head(g1, g2, g3, p1, p2, p3):
    w = jnp.zeros((3 * G, 128), jnp.float32)
    w = w.at[:G, 0].set(p1['pred_W'][0])
    w = w.at[G:2 * G, 1].set(p2['pred_W'][0])
    w = w.at[2 * G:, 2].set(p3['pred_W'][0])
    b = jnp.zeros((1, 128), jnp.float32)
    b = b.at[0, 0].set(p1['pred_b'][0]).at[0, 1].set(p2['pred_b'][0]).at[0, 2].set(p3['pred_b'][0])
    o = pl.pallas_call(
        _head_body,
        out_shape=jax.ShapeDtypeStruct((NUM_GRAPHS, 128), jnp.float32),
    )(g1, g2, g3, w, b)
    return o[:, :3]
